# baseline (speedup 1.0000x reference)
import functools

import jax
import jax.numpy as jnp
from jax import lax
from jax.experimental import pallas as pl
from jax.experimental.pallas import tpu as pltpu
from jax.experimental.pallas import tpu_sc as plsc

_BQ = 16384


def _bf16_round_bits(u):
    return (u + jnp.uint32(0x7FFF) + ((u >> 16) & jnp.uint32(1))) >> 16


def _tc_repack(tableT):
    D, V = tableT.shape
    nblk = (V + 4 * _BQ - 1) // (4 * _BQ)
    R_rows = nblk * _BQ

    def body(t_ref, out_ref):
        blk = t_ref[...]
        r0 = jax.lax.broadcasted_iota(jnp.int32, (D, D), 0)
        c0 = jax.lax.broadcasted_iota(jnp.int32, (D, D), 1)
        eye = (r0 == c0).astype(jnp.float32)
        dn = (((0,), (0,)), ((), ()))

        def halfpack(lo_slice, hi_slice):
            tl = jax.lax.dot_general(lo_slice, eye, dn,
                                     preferred_element_type=jnp.float32)
            th = jax.lax.dot_general(hi_slice, eye, dn,
                                     preferred_element_type=jnp.float32)
            bl = _bf16_round_bits(jax.lax.bitcast_convert_type(tl, jnp.uint32))
            bh = _bf16_round_bits(jax.lax.bitcast_convert_type(th, jnp.uint32))
            return jax.lax.bitcast_convert_type(bl | (bh << 16), jnp.int32)

        out_ref[:, :D] = halfpack(blk[:, :_BQ], blk[:, _BQ:2 * _BQ])
        out_ref[:, D:] = halfpack(blk[:, 2 * _BQ:3 * _BQ], blk[:, 3 * _BQ:])

    return pl.pallas_call(
        body,
        grid=(nblk,),
        in_specs=[pl.BlockSpec((D, 4 * _BQ), lambda j: (0, j))],
        out_specs=pl.BlockSpec((_BQ, 2 * D), lambda j: (j, 0)),
        out_shape=jax.ShapeDtypeStruct((R_rows, 2 * D), jnp.int32),
    )(tableT)


def _sc_gather(idx_flat, table2):
    V2, D2 = table2.shape
    N = idx_flat.shape[0]
    info = plsc.get_sparse_core_info()
    NC, NS = info.num_cores, info.num_subcores
    NW = NC * NS
    per_w = N // NW
    CH = 512
    n_ch = per_w // CH
    mesh = plsc.VectorSubcoreMesh(core_axis_name="c", subcore_axis_name="s")

    @functools.partial(
        pl.kernel,
        mesh=mesh,
        out_type=jax.ShapeDtypeStruct((N, D2), jnp.int32),
        scratch_types=[
            pltpu.VMEM((CH,), jnp.int32),
            pltpu.VMEM((CH, D2), jnp.int32),
            pltpu.SemaphoreType.DMA,
        ],
    )
    def gather_k(idx_hbm, table_hbm, out_hbm, idx_v, rows_v, sem):
        wid = lax.axis_index("s") * NC + lax.axis_index("c")
        base = wid * per_w

        def body(i, carry):
            off = base + i * CH
            pltpu.sync_copy(idx_hbm.at[pl.ds(off, CH)], idx_v)
            pltpu.async_copy(table_hbm.at[idx_v], rows_v, sem).wait()
            pltpu.sync_copy(rows_v, out_hbm.at[pl.ds(off, CH)])
            return carry

        lax.fori_loop(0, n_ch, body, 0)

    return gather_k(idx_flat, table2)


def _tc_mlp(emb3, sel, W1, b1, W2, b2):
    B, L, D2 = emb3.shape
    D = D2 // 2
    H = W1.shape[1]
    E = W2.shape[1]
    SB = 256
    grid = (B // SB,)

    def body(emb_ref, sel_ref, w1_ref, b1_ref, w2_ref, b2_ref, out_ref):
        w1 = w1_ref[...]
        acc = None
        for l in range(L):
            lo = emb_ref[:, l, :D]
            hi = emb_ref[:, l, D:]
            s = sel_ref[:, l].reshape(lo.shape[0], 1)
            w = jnp.where(s >= 2, hi, lo)
            wb = jax.lax.bitcast_convert_type(w, jnp.uint32)
            odd = (s & 1) == 1
            bits = jnp.where(odd, wb & jnp.uint32(0xFFFF0000), wb << 16)
            e = jax.lax.bitcast_convert_type(bits, jnp.float32)
            z = jnp.dot(e, w1, preferred_element_type=jnp.float32)
            acc = z if acc is None else jnp.maximum(acc, z)
        h = jnp.maximum(acc + b1_ref[...], 0.0)
        out_ref[...] = (
            jnp.dot(h, w2_ref[...], preferred_element_type=jnp.float32) + b2_ref[...]
        )

    return pl.pallas_call(
        body,
        grid=grid,
        in_specs=[
            pl.BlockSpec((SB, L, D2), lambda i: (i, 0, 0)),
            pl.BlockSpec((SB, L), lambda i: (i, 0)),
            pl.BlockSpec((D, H), lambda i: (0, 0)),
            pl.BlockSpec((1, H), lambda i: (0, 0)),
            pl.BlockSpec((H, E), lambda i: (0, 0)),
            pl.BlockSpec((1, E), lambda i: (0, 0)),
        ],
        out_specs=pl.BlockSpec((SB, E), lambda i: (i, 0)),
        out_shape=jax.ShapeDtypeStruct((B, E), jnp.float32),
    )(emb3, sel, W1, b1.reshape(1, H), W2, b2.reshape(1, E))


def kernel(x, table, W1, b1, W2, b2):
    B, L = x.shape
    V, D = table.shape
    R = _tc_repack(table.T)
    idx = x.reshape(-1).astype(jnp.int32)
    j = idx // (4 * _BQ)
    q = idx % (4 * _BQ)
    row = j * _BQ + (q % _BQ)
    sel = (q // _BQ).reshape(B, L)
    emb = _sc_gather(row, R)
    return _tc_mlp(emb.reshape(B, L, 2 * D), sel, W1, b1, W2, b2)


# speedup vs baseline: 1.0856x; 1.0856x over previous
import functools

import jax
import jax.numpy as jnp
from jax import lax
from jax.experimental import pallas as pl
from jax.experimental.pallas import tpu as pltpu
from jax.experimental.pallas import tpu_sc as plsc

_BQ = 8192


def _bf16_round_bits(u):
    return (u + jnp.uint32(0x8000)) >> 16


def _tc_repack(tableT):
    D, V = tableT.shape
    nblk = (V + 4 * _BQ - 1) // (4 * _BQ)
    R_rows = nblk * _BQ

    def body(t_ref, out_ref):
        blk = t_ref[...]

        def halfpack(lo_slice, hi_slice):
            tl = jnp.transpose(lo_slice)
            th = jnp.transpose(hi_slice)
            bl = _bf16_round_bits(jax.lax.bitcast_convert_type(tl, jnp.uint32))
            bh = _bf16_round_bits(jax.lax.bitcast_convert_type(th, jnp.uint32))
            return jax.lax.bitcast_convert_type(bl | (bh << 16), jnp.int32)

        out_ref[:, :D] = halfpack(blk[:, :_BQ], blk[:, _BQ:2 * _BQ])
        out_ref[:, D:] = halfpack(blk[:, 2 * _BQ:3 * _BQ], blk[:, 3 * _BQ:])

    return pl.pallas_call(
        body,
        grid=(nblk,),
        in_specs=[pl.BlockSpec((D, 4 * _BQ), lambda j: (0, j))],
        out_specs=pl.BlockSpec((_BQ, 2 * D), lambda j: (j, 0)),
        out_shape=jax.ShapeDtypeStruct((R_rows, 2 * D), jnp.int32),
    )(tableT)


def _sc_gather(idx_flat, table2):
    V2, D2 = table2.shape
    N = idx_flat.shape[0]
    info = plsc.get_sparse_core_info()
    NC, NS = info.num_cores, info.num_subcores
    NW = NC * NS
    per_w = N // NW
    CH = 256
    n_ch = per_w // CH
    mesh = plsc.VectorSubcoreMesh(core_axis_name="c", subcore_axis_name="s")

    @functools.partial(
        pl.kernel,
        mesh=mesh,
        out_type=jax.ShapeDtypeStruct((N, D2), jnp.int32),
        scratch_types=[
            pltpu.VMEM((per_w,), jnp.int32),
            pltpu.VMEM((CH, D2), jnp.int32),
            pltpu.VMEM((CH, D2), jnp.int32),
            pltpu.SemaphoreType.DMA,
            pltpu.SemaphoreType.DMA,
            pltpu.SemaphoreType.DMA,
            pltpu.SemaphoreType.DMA,
        ],
    )
    def gather_k(idx_hbm, table_hbm, out_hbm, idx_v, r0, r1, g0, g1, w0, w1):
        wid = lax.axis_index("s") * NC + lax.axis_index("c")
        base = wid * per_w
        pltpu.sync_copy(idx_hbm.at[pl.ds(base, per_w)], idx_v)
        rows = (r0, r1)
        gsem = (g0, g1)
        wsem = (w0, w1)
        pltpu.async_copy(table_hbm.at[idx_v.at[pl.ds(0, CH)]], r0, g0)
        for i in range(n_ch):
            p = i % 2
            if i + 1 < n_ch:
                if i >= 1:
                    pltpu.make_async_copy(
                        rows[1 - p],
                        out_hbm.at[pl.ds(base + (i - 1) * CH, CH)],
                        wsem[1 - p],
                    ).wait()
                pltpu.async_copy(
                    table_hbm.at[idx_v.at[pl.ds((i + 1) * CH, CH)]],
                    rows[1 - p], gsem[1 - p],
                )
            pltpu.make_async_copy(
                table_hbm.at[idx_v.at[pl.ds(i * CH, CH)]], rows[p], gsem[p]
            ).wait()
            pltpu.async_copy(
                rows[p], out_hbm.at[pl.ds(base + i * CH, CH)], wsem[p]
            )
        for i in range(max(n_ch - 2, 0), n_ch):
            p = i % 2
            pltpu.make_async_copy(
                rows[p], out_hbm.at[pl.ds(base + i * CH, CH)], wsem[p]
            ).wait()

    return gather_k(idx_flat, table2)


def _tc_mlp(emb3, sel, W1, b1, W2, b2):
    B, L, D2 = emb3.shape
    D = D2 // 2
    H = W1.shape[1]
    E = W2.shape[1]
    SB = 256
    grid = (B // SB,)

    def body(emb_ref, sel_ref, w1_ref, b1_ref, w2_ref, b2_ref, out_ref):
        w1 = w1_ref[...]
        acc = None
        for l in range(L):
            lo = emb_ref[:, l, :D]
            hi = emb_ref[:, l, D:]
            s = sel_ref[:, l].reshape(lo.shape[0], 1)
            w = jnp.where(s >= 2, hi, lo)
            wb = jax.lax.bitcast_convert_type(w, jnp.uint32)
            odd = (s & 1) == 1
            bits = jnp.where(odd, wb & jnp.uint32(0xFFFF0000), wb << 16)
            e = jax.lax.bitcast_convert_type(bits, jnp.float32)
            z = jnp.dot(e, w1, preferred_element_type=jnp.float32)
            acc = z if acc is None else jnp.maximum(acc, z)
        h = jnp.maximum(acc + b1_ref[...], 0.0)
        out_ref[...] = (
            jnp.dot(h, w2_ref[...], preferred_element_type=jnp.float32) + b2_ref[...]
        )

    return pl.pallas_call(
        body,
        grid=grid,
        in_specs=[
            pl.BlockSpec((SB, L, D2), lambda i: (i, 0, 0)),
            pl.BlockSpec((SB, L), lambda i: (i, 0)),
            pl.BlockSpec((D, H), lambda i: (0, 0)),
            pl.BlockSpec((1, H), lambda i: (0, 0)),
            pl.BlockSpec((H, E), lambda i: (0, 0)),
            pl.BlockSpec((1, E), lambda i: (0, 0)),
        ],
        out_specs=pl.BlockSpec((SB, E), lambda i: (i, 0)),
        out_shape=jax.ShapeDtypeStruct((B, E), jnp.float32),
    )(emb3, sel, W1, b1.reshape(1, H), W2, b2.reshape(1, E))


def kernel(x, table, W1, b1, W2, b2):
    B, L = x.shape
    V, D = table.shape
    R = _tc_repack(table.T)
    idx = x.reshape(-1).astype(jnp.int32)
    j = idx // (4 * _BQ)
    q = idx % (4 * _BQ)
    row = j * _BQ + (q % _BQ)
    sel = (q // _BQ).reshape(B, L)
    NS_ = 4
    Bh = B // NS_
    Nh = Bh * L
    embs = [_sc_gather(row[k * Nh:(k + 1) * Nh], R) for k in range(NS_)]
    outs = [
        _tc_mlp(e.reshape(Bh, L, 2 * D), sel[k * Bh:(k + 1) * Bh],
                W1, b1, W2, b2)
        for k, e in enumerate(embs)
    ]
    return jnp.concatenate(outs, axis=0)
